# baseline (speedup 1.0000x reference)
import functools

import jax
import jax.numpy as jnp
from jax import lax
from jax.experimental import pallas as pl
from jax.experimental.pallas import tpu as pltpu
from jax.experimental.pallas import tpu_sc as plsc

DIM = 64
NC = 2
NS = 16
NW = NC * NS
BLK = 128


@functools.partial(jax.jit, static_argnames=("fields", "batch"))
def _gather(tbl128, idx, fields, batch):
    nblk = fields * (batch // BLK)
    blk_per_w = nblk // NW
    mesh = plsc.VectorSubcoreMesh(core_axis_name="c", subcore_axis_name="s")

    @functools.partial(
        pl.kernel,
        mesh=mesh,
        out_type=jax.ShapeDtypeStruct((fields, DIM, batch), jnp.float32),
        compiler_params=pltpu.CompilerParams(needs_layout_passes=False),
        scratch_types=[
            pltpu.VMEM((BLK,), jnp.int32),
            pltpu.VMEM((BLK, 128), jnp.float32),
            pltpu.VMEM((DIM, BLK), jnp.float32),
            pltpu.SemaphoreType.DMA,
        ],
    )
    def gather_k(tbl_hbm, idx_hbm, out_hbm, idx_v, rows_v, trans_v, sem_g):
        wid = lax.axis_index("s") * NC + lax.axis_index("c")
        g0 = wid * blk_per_w
        lanes = lax.iota(jnp.int32, 16)

        def body(i, carry):
            g = g0 + i
            f = g // (batch // BLK)
            bb = g % (batch // BLK)
            off = f * batch + bb * BLK
            pltpu.sync_copy(idx_hbm.at[pl.ds(off, BLK)], idx_v)
            pltpu.async_copy(tbl_hbm.at[idx_v], rows_v, sem_g).wait()

            def dloop(d, c2):
                dvec = jnp.full((16,), 0, jnp.int32) + d
                for j in range(BLK // 16):
                    ridx = lax.iota(jnp.int32, 16) + (16 * j)
                    vals = plsc.load_gather(rows_v, [ridx, dvec])
                    trans_v[d, pl.ds(16 * j, 16)] = vals
                return c2

            lax.fori_loop(0, DIM, dloop, 0)
            pltpu.sync_copy(trans_v, out_hbm.at[f, :, pl.ds(bb * BLK, BLK)])
            return carry

        lax.fori_loop(0, blk_per_w, body, 0)

    return gather_k(tbl128, idx)


def kernel(cat_ids, table):
    batch, fields = cat_ids.shape
    idx = cat_ids.T.reshape(batch * fields).astype(jnp.int32)
    tbl128 = jnp.pad(table, ((0, 0), (0, 128 - DIM)))
    out3 = _gather(tbl128, idx, fields, batch)
    return out3.transpose(2, 0, 1)
